# Initial kernel scaffold; baseline (speedup 1.0000x reference)
import functools

import jax
import jax.numpy as jnp
from jax import lax
from jax.experimental import pallas as pl
from jax.experimental.pallas import tpu as pltpu
from jax.experimental.pallas import tpu_sc as plsc

NC = 2
NS = 16
NW = NC * NS
L = 16
D = 128
DA = 144
K = 128
EBLK = 2048


def _t1_body(u2e_ref, w1a_ref, w1b_ref, b1_ref, pa_ref, pb_ref, aug_ref):
    x = u2e_ref[...]
    pa_ref[...] = jnp.dot(x, w1a_ref[...], preferred_element_type=jnp.float32)
    pb_ref[...] = (
        jnp.dot(x, w1b_ref[...], preferred_element_type=jnp.float32) + b1_ref[...]
    )
    one_col = (
        lax.broadcasted_iota(jnp.int32, (x.shape[0], DA - D), 1) == 0
    ).astype(jnp.float32)
    aug_ref[...] = jnp.concatenate([x, one_col], axis=1)


def _node_projections(u2e, W1, b1):
    n = u2e.shape[0]
    nblk = 1000
    return pl.pallas_call(
        _t1_body,
        grid=(n // nblk,),
        in_specs=[
            pl.BlockSpec((nblk, D), lambda i: (i, 0)),
            pl.BlockSpec((D, D), lambda i: (0, 0)),
            pl.BlockSpec((D, D), lambda i: (0, 0)),
            pl.BlockSpec((1, D), lambda i: (0, 0)),
        ],
        out_specs=[
            pl.BlockSpec((nblk, D), lambda i: (i, 0)),
            pl.BlockSpec((nblk, D), lambda i: (i, 0)),
            pl.BlockSpec((nblk, DA), lambda i: (i, 0)),
        ],
        out_shape=[
            jax.ShapeDtypeStruct((n, D), jnp.float32),
            jax.ShapeDtypeStruct((n, D), jnp.float32),
            jax.ShapeDtypeStruct((n, DA), jnp.float32),
        ],
    )(u2e, W1[:D], W1[D:], b1.reshape(1, D))


def _t2_body(e_real, h1_ref, w2_ref, b2_ref, w3_ref, lg_ref, gmax_ref):
    i = pl.program_id(0)
    h1 = jnp.maximum(h1_ref[...], 0.0)
    h2 = jnp.maximum(
        jnp.dot(h1, w2_ref[...], preferred_element_type=jnp.float32) + b2_ref[...],
        0.0,
    )
    lg = jnp.sum(h2 * w3_ref[...], axis=1, keepdims=True)
    rows = i * EBLK + lax.broadcasted_iota(jnp.int32, (EBLK, 1), 0)
    lg = jnp.where(rows < e_real, lg, -1e30)
    lg_ref[...] = lg
    m = jnp.max(lg)

    @pl.when(i == 0)
    def _():
        gmax_ref[0, 0] = m

    @pl.when(i != 0)
    def _():
        gmax_ref[0, 0] = jnp.maximum(gmax_ref[0, 0], m)


def _edge_logits(h1, W2, b2, W3, e_real):
    e_pad = h1.shape[0]
    return pl.pallas_call(
        functools.partial(_t2_body, e_real),
        grid=(e_pad // EBLK,),
        in_specs=[
            pl.BlockSpec((EBLK, D), lambda i: (i, 0)),
            pl.BlockSpec((D, D), lambda i: (0, 0)),
            pl.BlockSpec((1, D), lambda i: (0, 0)),
            pl.BlockSpec((1, D), lambda i: (0, 0)),
        ],
        out_specs=[
            pl.BlockSpec((EBLK, 1), lambda i: (i, 0)),
            pl.BlockSpec((1, 1), lambda i: (0, 0)),
        ],
        out_shape=[
            jax.ShapeDtypeStruct((e_pad, 1), jnp.float32),
            jax.ShapeDtypeStruct((1, 1), jnp.float32),
        ],
    )(h1, W2, b2.reshape(1, D), W3.reshape(1, D))


def _sc_mesh():
    return plsc.VectorSubcoreMesh(
        core_axis_name="c", subcore_axis_name="s", num_cores=NC, num_subcores=NS
    )


def _make_s1(e_pad, n):
    epw = e_pad // NW

    @functools.partial(
        pl.kernel,
        mesh=_sc_mesh(),
        out_type=jax.ShapeDtypeStruct((e_pad, D), jnp.float32),
        scratch_types=[
            pltpu.VMEM((K,), jnp.int32),
            pltpu.VMEM((K,), jnp.int32),
            pltpu.VMEM((K, D), jnp.float32),
            pltpu.VMEM((K, D), jnp.float32),
        ],
    )
    def s1(pa_hbm, pb_hbm, row_hbm, col_hbm, h1_hbm, row_v, col_v, a_v, b_v):
        c = lax.axis_index("c")
        s = lax.axis_index("s")
        base = (s * NC + c) * epw

        def chunk(i, _):
            off = base + i * K
            pltpu.sync_copy(row_hbm.at[pl.ds(off, K)], row_v)
            pltpu.sync_copy(col_hbm.at[pl.ds(off, K)], col_v)
            pltpu.sync_copy(pa_hbm.at[row_v], a_v)
            pltpu.sync_copy(pb_hbm.at[col_v], b_v)

            def edge(e, _):
                for j in range(D // L):
                    sl = pl.ds(j * L, L)
                    a_v[e, sl] = a_v[e, sl] + b_v[e, sl]
                return 0

            lax.fori_loop(0, K, edge, 0)
            pltpu.sync_copy(a_v, h1_hbm.at[pl.ds(off, K)])
            return 0

        lax.fori_loop(0, epw // K, chunk, 0)

    return s1


def _make_s2(e_pad, n):
    epw = e_pad // NW
    npw = -(-n // NS)

    def _rows_per_tile(s_static):
        lo = s_static * npw
        hi = min(n, lo + npw)
        return lo, hi - lo

    @functools.partial(
        pl.kernel,
        mesh=_sc_mesh(),
        out_type=[
            jax.ShapeDtypeStruct((n, DA), jnp.float32),
            jax.ShapeDtypeStruct((n, DA), jnp.float32),
        ],
        scratch_types=[
            pltpu.VMEM((K,), jnp.int32),
            pltpu.VMEM((K,), jnp.int32),
            pltpu.VMEM((K,), jnp.float32),
            pltpu.VMEM((K, DA), jnp.float32),
            pltpu.VMEM((L,), jnp.float32),
            pltpu.VMEM_SHARED((n, DA), jnp.float32),
        ],
    )
    def s2(aug_hbm, row_hbm, col_hbm, lg_hbm, gv_hbm, p0_hbm, p1_hbm,
           row_v, col_v, w_v, u_v, g_v, acc):
        c = lax.axis_index("c")
        s = lax.axis_index("s")
        base = (s * NC + c) * epw

        def zrow(r, _):
            for j in range(DA // L):
                u_v[r, pl.ds(j * L, L)] = jnp.zeros((L,), jnp.float32)
            return 0

        lax.fori_loop(0, K, zrow, 0)

        for s_static in range(NS):
            @pl.when(s == s_static)
            def _():
                lo, cnt = _rows_per_tile(s_static)
                full, rem = cnt // K, cnt % K
                for t in range(full):
                    pltpu.sync_copy(u_v, acc.at[pl.ds(lo + t * K, K)])
                if rem:
                    pltpu.sync_copy(
                        u_v.at[pl.ds(0, rem)], acc.at[pl.ds(lo + full * K, rem)]
                    )

        pltpu.sync_copy(gv_hbm, g_v)
        plsc.subcore_barrier()

        def chunk(i, _):
            off = base + i * K
            pltpu.sync_copy(row_hbm.at[pl.ds(off, K)], row_v)
            pltpu.sync_copy(col_hbm.at[pl.ds(off, K)], col_v)
            pltpu.sync_copy(lg_hbm.at[pl.ds(off, K)], w_v)
            g = g_v[...]

            def wexp(j, _):
                sl = pl.ds(j * L, L)
                w_v[sl] = jnp.exp(w_v[sl] - g)
                return 0

            lax.fori_loop(0, K // L, wexp, 0)
            pltpu.sync_copy(aug_hbm.at[row_v], u_v)

            def edge(e, _):
                wv = plsc.load_gather(w_v, [jnp.full((L,), e, jnp.int32)])
                for j in range(DA // L):
                    sl = pl.ds(j * L, L)
                    u_v[e, sl] = u_v[e, sl] * wv
                return 0

            lax.fori_loop(0, K, edge, 0)
            pltpu.sync_copy(u_v, acc.at[col_v], add=True)
            return 0

        lax.fori_loop(0, epw // K, chunk, 0)
        plsc.subcore_barrier()

        for s_static in range(NS):
            @pl.when(s == s_static)
            def _():
                lo, cnt = _rows_per_tile(s_static)

                @pl.when(c == 0)
                def _():
                    pltpu.sync_copy(acc.at[pl.ds(lo, cnt)], p0_hbm.at[pl.ds(lo, cnt)])

                @pl.when(c == 1)
                def _():
                    pltpu.sync_copy(acc.at[pl.ds(lo, cnt)], p1_hbm.at[pl.ds(lo, cnt)])

    return s2


def _make_s3(b):
    bpw = b // NW

    @functools.partial(
        pl.kernel,
        mesh=_sc_mesh(),
        out_type=jax.ShapeDtypeStruct((b, D), jnp.float32),
        scratch_types=[
            pltpu.VMEM((bpw,), jnp.int32),
            pltpu.VMEM((bpw, DA), jnp.float32),
            pltpu.VMEM((bpw, DA), jnp.float32),
            pltpu.VMEM((bpw, D), jnp.float32),
        ],
    )
    def s3(p0_hbm, p1_hbm, nodes_hbm, feat_hbm, idx_v, u0, u1, o_v):
        c = lax.axis_index("c")
        s = lax.axis_index("s")
        base = (s * NC + c) * bpw
        pltpu.sync_copy(nodes_hbm.at[pl.ds(base, bpw)], idx_v)
        pltpu.sync_copy(p0_hbm.at[idx_v], u0)
        pltpu.sync_copy(p1_hbm.at[idx_v], u1)

        def node(r, _):
            ri = jnp.full((L,), r, jnp.int32)
            di = jnp.full((L,), D, jnp.int32)
            dv = plsc.load_gather(u0, [ri, di]) + plsc.load_gather(u1, [ri, di])
            rv = jnp.where(dv > 0.0, 1.0 / dv, 0.0)
            for j in range(D // L):
                sl = pl.ds(j * L, L)
                o_v[r, sl] = (u0[r, sl] + u1[r, sl]) * rv
            return 0

        lax.fori_loop(0, bpw, node, 0)
        pltpu.sync_copy(o_v, feat_hbm.at[pl.ds(base, bpw)])

    return s3


def kernel(nodes, row_idxs, col_idxs, u2e_weight, W1, b1, W2, b2, W3, b3):
    n = u2e_weight.shape[0]
    e = row_idxs.shape[0]
    b = nodes.shape[0]

    row = row_idxs.astype(jnp.int32)
    col = col_idxs.astype(jnp.int32)
    nds = nodes.astype(jnp.int32)

    quantum = 4096
    e_pad = -(-e // quantum) * quantum
    rowp = jnp.pad(row, (0, e_pad - e))
    colp = jnp.pad(col, (0, e_pad - e))

    pa, pb, aug = _node_projections(u2e_weight, W1, b1)
    h1 = _make_s1(e_pad, n)(pa, pb, rowp, colp)
    lg, gmax = _edge_logits(h1, W2, b2, W3, e)
    gv = jnp.broadcast_to(gmax.reshape(1), (L,))
    p0, p1 = _make_s2(e_pad, n)(aug, rowp, colp, lg.reshape(e_pad), gv)
    feat = _make_s3(b)(p0, p1, nds)
    return feat


# speedup vs baseline: 4.2438x; 4.2438x over previous
import functools

import jax
import jax.numpy as jnp
from jax import lax
from jax.experimental import pallas as pl
from jax.experimental.pallas import tpu as pltpu
from jax.experimental.pallas import tpu_sc as plsc

NC = 2
NS = 16
NW = NC * NS
L = 16
D = 128
DA = 144
K = 128
EBLK = 2048


def _t1_body(u2e_ref, w1a_ref, w1b_ref, b1_ref, pa_ref, pb_ref, aug_ref):
    x = u2e_ref[...]
    pa_ref[...] = jnp.dot(x, w1a_ref[...], preferred_element_type=jnp.float32)
    pb_ref[...] = (
        jnp.dot(x, w1b_ref[...], preferred_element_type=jnp.float32) + b1_ref[...]
    )
    one_col = (
        lax.broadcasted_iota(jnp.int32, (x.shape[0], DA - D), 1) == 0
    ).astype(jnp.float32)
    aug_ref[...] = jnp.concatenate([x, one_col], axis=1)


def _node_projections(u2e, W1, b1):
    n = u2e.shape[0]
    nblk = 1000
    return pl.pallas_call(
        _t1_body,
        grid=(n // nblk,),
        in_specs=[
            pl.BlockSpec((nblk, D), lambda i: (i, 0)),
            pl.BlockSpec((D, D), lambda i: (0, 0)),
            pl.BlockSpec((D, D), lambda i: (0, 0)),
            pl.BlockSpec((1, D), lambda i: (0, 0)),
        ],
        out_specs=[
            pl.BlockSpec((nblk, D), lambda i: (i, 0)),
            pl.BlockSpec((nblk, D), lambda i: (i, 0)),
            pl.BlockSpec((nblk, DA), lambda i: (i, 0)),
        ],
        out_shape=[
            jax.ShapeDtypeStruct((n, D), jnp.float32),
            jax.ShapeDtypeStruct((n, D), jnp.float32),
            jax.ShapeDtypeStruct((n, DA), jnp.float32),
        ],
    )(u2e, W1[:D], W1[D:], b1.reshape(1, D))


def _t2_body(e_real, h1_ref, w2_ref, b2_ref, w3_ref, lg_ref, gmax_ref):
    i = pl.program_id(0)
    h1 = jnp.maximum(h1_ref[...], 0.0)
    h2 = jnp.maximum(
        jnp.dot(h1, w2_ref[...], preferred_element_type=jnp.float32) + b2_ref[...],
        0.0,
    )
    lg = jnp.sum(h2 * w3_ref[...], axis=1, keepdims=True)
    rows = i * EBLK + lax.broadcasted_iota(jnp.int32, (EBLK, 1), 0)
    lg = jnp.where(rows < e_real, lg, -1e30)
    lg_ref[...] = lg
    m = jnp.max(lg, axis=0, keepdims=True)

    @pl.when(i == 0)
    def _():
        gmax_ref[...] = m

    @pl.when(i != 0)
    def _():
        gmax_ref[...] = jnp.maximum(gmax_ref[...], m)


def _edge_logits(h1, W2, b2, W3, e_real):
    e_pad = h1.shape[0]
    return pl.pallas_call(
        functools.partial(_t2_body, e_real),
        grid=(e_pad // EBLK,),
        in_specs=[
            pl.BlockSpec((EBLK, D), lambda i: (i, 0)),
            pl.BlockSpec((D, D), lambda i: (0, 0)),
            pl.BlockSpec((1, D), lambda i: (0, 0)),
            pl.BlockSpec((1, D), lambda i: (0, 0)),
        ],
        out_specs=[
            pl.BlockSpec((EBLK, 1), lambda i: (i, 0)),
            pl.BlockSpec((1, 1), lambda i: (0, 0)),
        ],
        out_shape=[
            jax.ShapeDtypeStruct((e_pad, 1), jnp.float32),
            jax.ShapeDtypeStruct((1, 1), jnp.float32),
        ],
    )(h1, W2, b2.reshape(1, D), W3.reshape(1, D))


def _sc_mesh():
    return plsc.VectorSubcoreMesh(
        core_axis_name="c", subcore_axis_name="s", num_cores=NC, num_subcores=NS
    )


_SC_PARAMS = pltpu.CompilerParams(
    use_tc_tiling_on_sc=False, needs_layout_passes=False
)


def _make_s1(e_pad, n):
    epw = e_pad // NW

    @functools.partial(
        pl.kernel,
        mesh=_sc_mesh(),
        compiler_params=_SC_PARAMS,
        out_type=jax.ShapeDtypeStruct((e_pad, D), jnp.float32),
        scratch_types=[
            pltpu.VMEM((K,), jnp.int32),
            pltpu.VMEM((K,), jnp.int32),
            pltpu.VMEM((K, D), jnp.float32),
            pltpu.VMEM((K, D), jnp.float32),
        ],
    )
    def s1(pa_hbm, pb_hbm, row_hbm, col_hbm, h1_hbm, row_v, col_v, a_v, b_v):
        c = lax.axis_index("c")
        s = lax.axis_index("s")
        base = (s * NC + c) * epw

        def chunk(i, _):
            off = base + i * K
            pltpu.sync_copy(row_hbm.at[pl.ds(off, K)], row_v)
            pltpu.sync_copy(col_hbm.at[pl.ds(off, K)], col_v)
            pltpu.sync_copy(pa_hbm.at[row_v], a_v)
            pltpu.sync_copy(pb_hbm.at[col_v], b_v)

            def edge(e, _):
                for j in range(D // L):
                    sl = pl.ds(j * L, L)
                    a_v[e, sl] = a_v[e, sl] + b_v[e, sl]
                return 0

            lax.fori_loop(0, K, edge, 0)
            pltpu.sync_copy(a_v, h1_hbm.at[pl.ds(off, K)])
            return 0

        lax.fori_loop(0, epw // K, chunk, 0)

    return s1


def _make_s2(e_pad, n):
    epw = e_pad // NW
    npw = -(-n // NS)

    def _rows_per_tile(s_static):
        lo = s_static * npw
        hi = min(n, lo + npw)
        return lo, hi - lo

    @functools.partial(
        pl.kernel,
        mesh=_sc_mesh(),
        compiler_params=_SC_PARAMS,
        out_type=[
            jax.ShapeDtypeStruct((n, DA), jnp.float32),
            jax.ShapeDtypeStruct((n, DA), jnp.float32),
        ],
        scratch_types=[
            pltpu.VMEM((K,), jnp.int32),
            pltpu.VMEM((K,), jnp.int32),
            pltpu.VMEM((K,), jnp.float32),
            pltpu.VMEM((K, DA), jnp.float32),
            pltpu.VMEM((L,), jnp.float32),
            pltpu.VMEM_SHARED((n, DA), jnp.float32),
        ],
    )
    def s2(aug_hbm, row_hbm, col_hbm, lg_hbm, gv_hbm, p0_hbm, p1_hbm,
           row_v, col_v, w_v, u_v, g_v, acc):
        c = lax.axis_index("c")
        s = lax.axis_index("s")
        base = (s * NC + c) * epw

        def zrow(r, _):
            for j in range(DA // L):
                u_v[r, pl.ds(j * L, L)] = jnp.zeros((L,), jnp.float32)
            return 0

        lax.fori_loop(0, K, zrow, 0)

        for s_static in range(NS):
            @pl.when(s == s_static)
            def _():
                lo, cnt = _rows_per_tile(s_static)
                full, rem = cnt // K, cnt % K
                for t in range(full):
                    pltpu.sync_copy(u_v, acc.at[pl.ds(lo + t * K, K)])
                if rem:
                    pltpu.sync_copy(
                        u_v.at[pl.ds(0, rem)], acc.at[pl.ds(lo + full * K, rem)]
                    )

        pltpu.sync_copy(gv_hbm, g_v)
        plsc.subcore_barrier()

        def chunk(i, _):
            off = base + i * K
            pltpu.sync_copy(row_hbm.at[pl.ds(off, K)], row_v)
            pltpu.sync_copy(col_hbm.at[pl.ds(off, K)], col_v)
            pltpu.sync_copy(lg_hbm.at[pl.ds(off, K)], w_v)
            g = g_v[...]

            def wexp(j, _):
                sl = pl.ds(j * L, L)
                w_v[sl] = jnp.exp(w_v[sl] - g)
                return 0

            lax.fori_loop(0, K // L, wexp, 0)
            pltpu.sync_copy(aug_hbm.at[row_v], u_v)

            def edge(e, _):
                wv = plsc.load_gather(w_v, [jnp.full((L,), e, jnp.int32)])
                for j in range(DA // L):
                    sl = pl.ds(j * L, L)
                    u_v[e, sl] = u_v[e, sl] * wv
                return 0

            lax.fori_loop(0, K, edge, 0)
            pltpu.sync_copy(u_v, acc.at[col_v], add=True)
            return 0

        lax.fori_loop(0, epw // K, chunk, 0)
        plsc.subcore_barrier()

        for s_static in range(NS):
            @pl.when(s == s_static)
            def _():
                lo, cnt = _rows_per_tile(s_static)

                @pl.when(c == 0)
                def _():
                    pltpu.sync_copy(acc.at[pl.ds(lo, cnt)], p0_hbm.at[pl.ds(lo, cnt)])

                @pl.when(c == 1)
                def _():
                    pltpu.sync_copy(acc.at[pl.ds(lo, cnt)], p1_hbm.at[pl.ds(lo, cnt)])

    return s2


def _make_s3(b):
    bpw = b // NW

    @functools.partial(
        pl.kernel,
        mesh=_sc_mesh(),
        compiler_params=_SC_PARAMS,
        out_type=jax.ShapeDtypeStruct((b, D), jnp.float32),
        scratch_types=[
            pltpu.VMEM((bpw,), jnp.int32),
            pltpu.VMEM((bpw, DA), jnp.float32),
            pltpu.VMEM((bpw, DA), jnp.float32),
            pltpu.VMEM((bpw, D), jnp.float32),
        ],
    )
    def s3(p0_hbm, p1_hbm, nodes_hbm, feat_hbm, idx_v, u0, u1, o_v):
        c = lax.axis_index("c")
        s = lax.axis_index("s")
        base = (s * NC + c) * bpw
        pltpu.sync_copy(nodes_hbm.at[pl.ds(base, bpw)], idx_v)
        pltpu.sync_copy(p0_hbm.at[idx_v], u0)
        pltpu.sync_copy(p1_hbm.at[idx_v], u1)

        def node(r, _):
            ri = jnp.full((L,), r, jnp.int32)
            di = jnp.full((L,), D, jnp.int32)
            dv = plsc.load_gather(u0, [ri, di]) + plsc.load_gather(u1, [ri, di])
            rv = jnp.where(dv > 0.0, 1.0 / dv, 0.0)
            for j in range(D // L):
                sl = pl.ds(j * L, L)
                o_v[r, sl] = (u0[r, sl] + u1[r, sl]) * rv
            return 0

        lax.fori_loop(0, bpw, node, 0)
        pltpu.sync_copy(o_v, feat_hbm.at[pl.ds(base, bpw)])

    return s3


def kernel(nodes, row_idxs, col_idxs, u2e_weight, W1, b1, W2, b2, W3, b3):
    n = u2e_weight.shape[0]
    e = row_idxs.shape[0]
    b = nodes.shape[0]

    row = row_idxs.astype(jnp.int32)
    col = col_idxs.astype(jnp.int32)
    nds = nodes.astype(jnp.int32)

    quantum = 4096
    e_pad = -(-e // quantum) * quantum
    rowp = jnp.pad(row, (0, e_pad - e))
    colp = jnp.pad(col, (0, e_pad - e))

    pa, pb, aug = _node_projections(u2e_weight, W1, b1)
    h1 = _make_s1(e_pad, n)(pa, pb, rowp, colp)
    lg, gmax = _edge_logits(h1, W2, b2, W3, e)
    gv = jnp.broadcast_to(gmax.reshape(1), (L,))
    p0, p1 = _make_s2(e_pad, n)(aug, rowp, colp, lg.reshape(e_pad), gv)
    feat = _make_s3(b)(p0, p1, nds)
    return feat


# speedup vs baseline: 6.6249x; 1.5611x over previous
import functools

import jax
import jax.numpy as jnp
from jax import lax
from jax.experimental import pallas as pl
from jax.experimental.pallas import tpu as pltpu
from jax.experimental.pallas import tpu_sc as plsc

NC = 2
NS = 16
NW = NC * NS
L = 16
D = 128
DA = 144
K = 128
EBLK = 2048


def _t1_body(u2e_ref, w1a_ref, w1b_ref, b1_ref, pa_ref, pb_ref, aug_ref):
    x = u2e_ref[...]
    pa_ref[...] = jnp.dot(x, w1a_ref[...], preferred_element_type=jnp.float32)
    pb_ref[...] = (
        jnp.dot(x, w1b_ref[...], preferred_element_type=jnp.float32) + b1_ref[...]
    )
    one_col = (
        lax.broadcasted_iota(jnp.int32, (x.shape[0], DA - D), 1) == 0
    ).astype(jnp.float32)
    aug_ref[...] = jnp.concatenate([x, one_col], axis=1)


def _node_projections(u2e, W1, b1):
    n = u2e.shape[0]
    nblk = 1000
    return pl.pallas_call(
        _t1_body,
        grid=(n // nblk,),
        in_specs=[
            pl.BlockSpec((nblk, D), lambda i: (i, 0)),
            pl.BlockSpec((D, D), lambda i: (0, 0)),
            pl.BlockSpec((D, D), lambda i: (0, 0)),
            pl.BlockSpec((1, D), lambda i: (0, 0)),
        ],
        out_specs=[
            pl.BlockSpec((nblk, D), lambda i: (i, 0)),
            pl.BlockSpec((nblk, D), lambda i: (i, 0)),
            pl.BlockSpec((nblk, DA), lambda i: (i, 0)),
        ],
        out_shape=[
            jax.ShapeDtypeStruct((n, D), jnp.float32),
            jax.ShapeDtypeStruct((n, D), jnp.float32),
            jax.ShapeDtypeStruct((n, DA), jnp.float32),
        ],
    )(u2e, W1[:D], W1[D:], b1.reshape(1, D))


def _t2_body(e_real, h1_ref, w2_ref, b2_ref, w3_ref, lg_ref, gmax_ref):
    i = pl.program_id(0)
    h1 = jnp.maximum(h1_ref[...], 0.0)
    h2 = jnp.maximum(
        jnp.dot(h1, w2_ref[...], preferred_element_type=jnp.float32) + b2_ref[...],
        0.0,
    )
    lg = jnp.sum(h2 * w3_ref[...], axis=1, keepdims=True)
    rows = i * EBLK + lax.broadcasted_iota(jnp.int32, (EBLK, 1), 0)
    lg = jnp.where(rows < e_real, lg, -1e30)
    lg_ref[...] = lg
    m = jnp.max(lg, axis=0, keepdims=True)

    @pl.when(i == 0)
    def _():
        gmax_ref[...] = m

    @pl.when(i != 0)
    def _():
        gmax_ref[...] = jnp.maximum(gmax_ref[...], m)


def _edge_logits(h1, W2, b2, W3, e_real):
    e_pad = h1.shape[0]
    return pl.pallas_call(
        functools.partial(_t2_body, e_real),
        grid=(e_pad // EBLK,),
        in_specs=[
            pl.BlockSpec((EBLK, D), lambda i: (i, 0)),
            pl.BlockSpec((D, D), lambda i: (0, 0)),
            pl.BlockSpec((1, D), lambda i: (0, 0)),
            pl.BlockSpec((1, D), lambda i: (0, 0)),
        ],
        out_specs=[
            pl.BlockSpec((EBLK, 1), lambda i: (i, 0)),
            pl.BlockSpec((1, 1), lambda i: (0, 0)),
        ],
        out_shape=[
            jax.ShapeDtypeStruct((e_pad, 1), jnp.float32),
            jax.ShapeDtypeStruct((1, 1), jnp.float32),
        ],
    )(h1, W2, b2.reshape(1, D), W3.reshape(1, D))


def _sc_mesh():
    return plsc.VectorSubcoreMesh(
        core_axis_name="c", subcore_axis_name="s", num_cores=NC, num_subcores=NS
    )


_SC_PARAMS = pltpu.CompilerParams(
    use_tc_tiling_on_sc=False, needs_layout_passes=False
)


def _make_s1(e_pad, n):
    epw = e_pad // NW

    nch = epw // K

    @functools.partial(
        pl.kernel,
        mesh=_sc_mesh(),
        compiler_params=_SC_PARAMS,
        out_type=jax.ShapeDtypeStruct((e_pad, D), jnp.float32),
        scratch_types=[
            pltpu.VMEM((K,), jnp.int32), pltpu.VMEM((K,), jnp.int32),
            pltpu.VMEM((K,), jnp.int32), pltpu.VMEM((K,), jnp.int32),
            pltpu.VMEM((K, D), jnp.float32), pltpu.VMEM((K, D), jnp.float32),
            pltpu.VMEM((K, D), jnp.float32), pltpu.VMEM((K, D), jnp.float32),
            pltpu.VMEM((K, D), jnp.float32), pltpu.VMEM((K, D), jnp.float32),
            pltpu.SemaphoreType.DMA, pltpu.SemaphoreType.DMA,
            pltpu.SemaphoreType.DMA, pltpu.SemaphoreType.DMA,
            pltpu.SemaphoreType.DMA, pltpu.SemaphoreType.DMA,
        ],
    )
    def s1(pa_hbm, pb_hbm, row_hbm, col_hbm, h1_hbm,
           r0, r1, c0, c1, a0, a1, b0, b1, o0, o1,
           si0, si1, sg0, sg1, sw0, sw1):
        c = lax.axis_index("c")
        s = lax.axis_index("s")
        base = (s * NC + c) * epw
        ring = ((r0, c0, a0, b0, o0, si0, sg0, sw0),
                (r1, c1, a1, b1, o1, si1, sg1, sw1))

        pltpu.sync_copy(row_hbm.at[pl.ds(base, K)], r0)
        pltpu.sync_copy(col_hbm.at[pl.ds(base, K)], c0)
        pltpu.async_copy(row_hbm.at[pl.ds(base + K, K)], r1, si1)
        pltpu.async_copy(col_hbm.at[pl.ds(base + K, K)], c1, si1)
        pltpu.async_copy(pa_hbm.at[r0], a0, sg0)
        pltpu.async_copy(pb_hbm.at[c0], b0, sg0)

        def pair(t, _):
            for p in (0, 1):
                ch = 2 * t + p
                rp, cp, ap, bp, op, sip, sgp, swp = ring[p]
                rq, cq, aq, bq, oq, siq, sgq, swq = ring[1 - p]

                @pl.when(ch + 1 < nch)
                def _():
                    pltpu.make_async_copy(row_hbm.at[pl.ds(base, K)], rq, siq).wait()
                    pltpu.make_async_copy(col_hbm.at[pl.ds(base, K)], cq, siq).wait()
                    pltpu.async_copy(pa_hbm.at[rq], aq, sgq)
                    pltpu.async_copy(pb_hbm.at[cq], bq, sgq)

                pltpu.make_async_copy(pa_hbm.at[rp], ap, sgp).wait()
                pltpu.make_async_copy(pb_hbm.at[cp], bp, sgp).wait()

                @pl.when(ch >= 2)
                def _():
                    pltpu.make_async_copy(op, h1_hbm.at[pl.ds(base, K)], swp).wait()

                def edge(e, _):
                    for j in range(D // L):
                        sl = pl.ds(j * L, L)
                        op[e, sl] = ap[e, sl] + bp[e, sl]
                    return 0

                lax.fori_loop(0, K, edge, 0)
                pltpu.async_copy(op, h1_hbm.at[pl.ds(base + ch * K, K)], swp)

                @pl.when(ch + 2 < nch)
                def _():
                    off2 = base + (ch + 2) * K
                    pltpu.async_copy(row_hbm.at[pl.ds(off2, K)], rp, sip)
                    pltpu.async_copy(col_hbm.at[pl.ds(off2, K)], cp, sip)
            return 0

        lax.fori_loop(0, nch // 2, pair, 0)
        pltpu.make_async_copy(o0, h1_hbm.at[pl.ds(base, K)], sw0).wait()
        pltpu.make_async_copy(o1, h1_hbm.at[pl.ds(base, K)], sw1).wait()

    return s1


def _make_s2(e_pad, n):
    epw = e_pad // NW
    npw = -(-n // NS)

    def _rows_per_tile(s_static):
        lo = s_static * npw
        hi = min(n, lo + npw)
        return lo, hi - lo

    @functools.partial(
        pl.kernel,
        mesh=_sc_mesh(),
        compiler_params=_SC_PARAMS,
        out_type=[
            jax.ShapeDtypeStruct((n, DA), jnp.float32),
            jax.ShapeDtypeStruct((n, DA), jnp.float32),
        ],
        scratch_types=[
            pltpu.VMEM((K,), jnp.int32), pltpu.VMEM((K,), jnp.int32),
            pltpu.VMEM((K,), jnp.int32), pltpu.VMEM((K,), jnp.int32),
            pltpu.VMEM((K,), jnp.float32), pltpu.VMEM((K,), jnp.float32),
            pltpu.VMEM((K, DA), jnp.float32), pltpu.VMEM((K, DA), jnp.float32),
            pltpu.VMEM((L,), jnp.float32),
            pltpu.VMEM_SHARED((n, DA), jnp.float32),
            pltpu.SemaphoreType.DMA, pltpu.SemaphoreType.DMA,
            pltpu.SemaphoreType.DMA, pltpu.SemaphoreType.DMA,
        ],
    )
    def s2(aug_hbm, row_hbm, col_hbm, lg_hbm, gv_hbm, p0_hbm, p1_hbm,
           r0, r1, c0, c1, w0, w1, u0, u1, g_v, acc,
           si0, si1, sg0, sg1):
        c = lax.axis_index("c")
        s = lax.axis_index("s")
        base = (s * NC + c) * epw
        nch = epw // K
        ring = ((r0, c0, w0, u0, si0, sg0), (r1, c1, w1, u1, si1, sg1))

        def zrow(r, _):
            for j in range(DA // L):
                u0[r, pl.ds(j * L, L)] = jnp.zeros((L,), jnp.float32)
            return 0

        lax.fori_loop(0, K, zrow, 0)

        for s_static in range(NS):
            @pl.when(s == s_static)
            def _():
                lo, cnt = _rows_per_tile(s_static)
                full, rem = cnt // K, cnt % K
                for t in range(full):
                    pltpu.sync_copy(u0, acc.at[pl.ds(lo + t * K, K)])
                if rem:
                    pltpu.sync_copy(
                        u0.at[pl.ds(0, rem)], acc.at[pl.ds(lo + full * K, rem)]
                    )

        pltpu.sync_copy(gv_hbm, g_v)
        plsc.subcore_barrier()

        pltpu.sync_copy(row_hbm.at[pl.ds(base, K)], r0)
        pltpu.sync_copy(col_hbm.at[pl.ds(base, K)], c0)
        pltpu.sync_copy(lg_hbm.at[pl.ds(base, K)], w0)
        pltpu.async_copy(row_hbm.at[pl.ds(base + K, K)], r1, si1)
        pltpu.async_copy(col_hbm.at[pl.ds(base + K, K)], c1, si1)
        pltpu.async_copy(lg_hbm.at[pl.ds(base + K, K)], w1, si1)
        pltpu.async_copy(aug_hbm.at[r0], u0, sg0)

        def pair(t, _):
            for p in (0, 1):
                ch = 2 * t + p
                rp, cp, wp, up, sip, sgp = ring[p]
                rq, cq, wq, uq, siq, sgq = ring[1 - p]

                @pl.when(ch + 1 < nch)
                def _():
                    pltpu.make_async_copy(row_hbm.at[pl.ds(base, K)], rq, siq).wait()
                    pltpu.make_async_copy(col_hbm.at[pl.ds(base, K)], cq, siq).wait()
                    pltpu.make_async_copy(lg_hbm.at[pl.ds(base, K)], wq, siq).wait()
                    pltpu.async_copy(aug_hbm.at[rq], uq, sgq)

                pltpu.make_async_copy(aug_hbm.at[rp], up, sgp).wait()
                g = g_v[...]

                def wexp(j, _):
                    sl = pl.ds(j * L, L)
                    wp[sl] = jnp.exp(wp[sl] - g)
                    return 0

                lax.fori_loop(0, K // L, wexp, 0)

                def edge(e, _):
                    wv = plsc.load_gather(wp, [jnp.full((L,), e, jnp.int32)])
                    for j in range(DA // L):
                        sl = pl.ds(j * L, L)
                        up[e, sl] = up[e, sl] * wv
                    return 0

                lax.fori_loop(0, K, edge, 0)
                pltpu.sync_copy(up, acc.at[cp], add=True)

                @pl.when(ch + 2 < nch)
                def _():
                    off2 = base + (ch + 2) * K
                    pltpu.async_copy(row_hbm.at[pl.ds(off2, K)], rp, sip)
                    pltpu.async_copy(col_hbm.at[pl.ds(off2, K)], cp, sip)
                    pltpu.async_copy(lg_hbm.at[pl.ds(off2, K)], wp, sip)
            return 0

        lax.fori_loop(0, nch // 2, pair, 0)
        plsc.subcore_barrier()

        for s_static in range(NS):
            @pl.when(s == s_static)
            def _():
                lo, cnt = _rows_per_tile(s_static)

                @pl.when(c == 0)
                def _():
                    pltpu.sync_copy(acc.at[pl.ds(lo, cnt)], p0_hbm.at[pl.ds(lo, cnt)])

                @pl.when(c == 1)
                def _():
                    pltpu.sync_copy(acc.at[pl.ds(lo, cnt)], p1_hbm.at[pl.ds(lo, cnt)])

    return s2


def _make_s3(b):
    bpw = b // NW

    @functools.partial(
        pl.kernel,
        mesh=_sc_mesh(),
        compiler_params=_SC_PARAMS,
        out_type=jax.ShapeDtypeStruct((b, D), jnp.float32),
        scratch_types=[
            pltpu.VMEM((bpw,), jnp.int32),
            pltpu.VMEM((bpw, DA), jnp.float32),
            pltpu.VMEM((bpw, DA), jnp.float32),
            pltpu.VMEM((bpw, D), jnp.float32),
        ],
    )
    def s3(p0_hbm, p1_hbm, nodes_hbm, feat_hbm, idx_v, u0, u1, o_v):
        c = lax.axis_index("c")
        s = lax.axis_index("s")
        base = (s * NC + c) * bpw
        pltpu.sync_copy(nodes_hbm.at[pl.ds(base, bpw)], idx_v)
        pltpu.sync_copy(p0_hbm.at[idx_v], u0)
        pltpu.sync_copy(p1_hbm.at[idx_v], u1)

        def node(r, _):
            ri = jnp.full((L,), r, jnp.int32)
            di = jnp.full((L,), D, jnp.int32)
            dv = plsc.load_gather(u0, [ri, di]) + plsc.load_gather(u1, [ri, di])
            rv = jnp.where(dv > 0.0, 1.0 / dv, 0.0)
            for j in range(D // L):
                sl = pl.ds(j * L, L)
                o_v[r, sl] = (u0[r, sl] + u1[r, sl]) * rv
            return 0

        lax.fori_loop(0, bpw, node, 0)
        pltpu.sync_copy(o_v, feat_hbm.at[pl.ds(base, bpw)])

    return s3


def kernel(nodes, row_idxs, col_idxs, u2e_weight, W1, b1, W2, b2, W3, b3):
    n = u2e_weight.shape[0]
    e = row_idxs.shape[0]
    b = nodes.shape[0]

    row = row_idxs.astype(jnp.int32)
    col = col_idxs.astype(jnp.int32)
    nds = nodes.astype(jnp.int32)

    quantum = 4096
    e_pad = -(-e // quantum) * quantum
    rowp = jnp.pad(row, (0, e_pad - e))
    colp = jnp.pad(col, (0, e_pad - e))

    pa, pb, aug = _node_projections(u2e_weight, W1, b1)
    h1 = _make_s1(e_pad, n)(pa, pb, rowp, colp)
    lg, gmax = _edge_logits(h1, W2, b2, W3, e)
    gv = jnp.broadcast_to(gmax.reshape(1), (L,))
    p0, p1 = _make_s2(e_pad, n)(aug, rowp, colp, lg.reshape(e_pad), gv)
    feat = _make_s3(b)(p0, p1, nds)
    return feat
